# baseline (speedup 1.0000x reference)
import functools

import jax
import jax.numpy as jnp
from jax.experimental import pallas as pl


def _matmul_bias_kernel(x_ref, w_ref, b_ref, o_ref, *, relu):
    acc = jnp.dot(x_ref[...], w_ref[...], preferred_element_type=jnp.float32)
    acc = acc + b_ref[...]
    if relu:
        acc = jnp.maximum(acc, 0.0)
    o_ref[...] = acc


def _matmul_bias(x, w, b, relu=False, bm=1024):
    m, k = x.shape
    n = w.shape[1]
    grid = (pl.cdiv(m, bm),)
    return pl.pallas_call(
        functools.partial(_matmul_bias_kernel, relu=relu),
        grid=grid,
        in_specs=[
            pl.BlockSpec((bm, k), lambda i: (i, 0)),
            pl.BlockSpec((k, n), lambda i: (0, 0)),
            pl.BlockSpec((1, n), lambda i: (0, 0)),
        ],
        out_specs=pl.BlockSpec((bm, n), lambda i: (i, 0)),
        out_shape=jax.ShapeDtypeStruct((m, n), jnp.float32),
    )(x, w, b.reshape(1, n))


def kernel(h, s_l, scores, W_att, b_att, W1, b1, W2, b2):
    n = h.shape[0]
    number_up = n // 4
    number_down = n - number_up
    M_i = 5

    order = jnp.flip(jnp.argsort(scores))
    up_idx = order[:number_up]
    up_points = jnp.zeros((n,), dtype=bool).at[up_idx].set(True)
    group_order = jnp.argsort(up_points, stable=True)
    nodes_down = group_order[:number_down]
    nodes_up = group_order[number_down:]

    down_pos = s_l[nodes_down]
    up_pos = s_l[nodes_up]
    d2 = jnp.sum((down_pos[:, None, :] - up_pos[None, :, :]) ** 2, axis=-1)
    _, neigh_indices = jax.lax.top_k(-d2, M_i)
    j = nodes_up[neigh_indices].reshape(-1)
    i = jnp.repeat(nodes_down, M_i)

    hs = jnp.concatenate([h, s_l], axis=1)
    feats = _matmul_bias(hs, W_att, b_att, bm=1000)

    diff = feats[i] - feats[j]
    m = _matmul_bias(diff, W1, b1, relu=True, bm=512)
    agg = jax.ops.segment_sum(m, j, num_segments=n)
    cnt = jax.ops.segment_sum(jnp.ones(j.shape, dtype=jnp.float32), j, num_segments=n)
    agg = agg / jnp.maximum(cnt, 1.0)[:, None]
    feats = feats + _matmul_bias(agg, W2, b2, bm=1000)
    return feats, up_points, i, j


# speedup vs baseline: 1.2517x; 1.2517x over previous
import functools

import jax
import jax.numpy as jnp
from jax.experimental import pallas as pl


def _matmul_bias_kernel(x_ref, w_ref, b_ref, o_ref, *, relu):
    acc = jnp.dot(x_ref[...], w_ref[...], preferred_element_type=jnp.float32)
    acc = acc + b_ref[...]
    if relu:
        acc = jnp.maximum(acc, 0.0)
    o_ref[...] = acc


def _matmul_bias(x, w, b, relu=False, bm=1024):
    m, k = x.shape
    n = w.shape[1]
    grid = (pl.cdiv(m, bm),)
    return pl.pallas_call(
        functools.partial(_matmul_bias_kernel, relu=relu),
        grid=grid,
        in_specs=[
            pl.BlockSpec((bm, k), lambda i: (i, 0)),
            pl.BlockSpec((k, n), lambda i: (0, 0)),
            pl.BlockSpec((1, n), lambda i: (0, 0)),
        ],
        out_specs=pl.BlockSpec((bm, n), lambda i: (i, 0)),
        out_shape=jax.ShapeDtypeStruct((m, n), jnp.float32),
    )(x, w, b.reshape(1, n))


def _knn_kernel(dp_ref, upt_ref, o_ref, *, nu, m_i):
    dx = dp_ref[:, 0:1]
    dy = dp_ref[:, 1:2]
    dz = dp_ref[:, 2:3]
    ux = upt_ref[0:1, :]
    uy = upt_ref[1:2, :]
    uz = upt_ref[2:3, :]
    d2 = (dx - ux) ** 2 + (dy - uy) ** 2
    d2 = d2 + (dz - uz) ** 2
    bm = d2.shape[0]
    iota = jax.lax.broadcasted_iota(jnp.int32, (bm, nu), 1)
    for k in range(m_i):
        mv = jnp.min(d2, axis=1, keepdims=True)
        sel = jnp.where(d2 == mv, iota, nu)
        mi = jnp.min(sel, axis=1, keepdims=True)
        o_ref[:, k : k + 1] = mi
        d2 = jnp.where(iota == mi, jnp.inf, d2)


def _knn(down_pos, up_pos_t, m_i, bm=768):
    nd = down_pos.shape[0]
    nu = up_pos_t.shape[1]
    return pl.pallas_call(
        functools.partial(_knn_kernel, nu=nu, m_i=m_i),
        grid=(pl.cdiv(nd, bm),),
        in_specs=[
            pl.BlockSpec((bm, 3), lambda i: (i, 0)),
            pl.BlockSpec((3, nu), lambda i: (0, 0)),
        ],
        out_specs=pl.BlockSpec((bm, m_i), lambda i: (i, 0)),
        out_shape=jax.ShapeDtypeStruct((nd, m_i), jnp.int32),
    )(down_pos, up_pos_t)


def kernel(h, s_l, scores, W_att, b_att, W1, b1, W2, b2):
    n = h.shape[0]
    number_up = n // 4
    number_down = n - number_up
    M_i = 5

    order = jnp.flip(jnp.argsort(scores))
    up_idx = order[:number_up]
    up_points = jnp.zeros((n,), dtype=bool).at[up_idx].set(True)
    group_order = jnp.argsort(up_points, stable=True)
    nodes_down = group_order[:number_down]
    nodes_up = group_order[number_down:]

    down_pos = s_l[nodes_down]
    up_pos_t = s_l[nodes_up].T
    neigh = _knn(down_pos, up_pos_t, M_i)
    jpos = neigh.reshape(-1)
    j = nodes_up[jpos]
    i = jnp.repeat(nodes_down, M_i)

    hs = jnp.concatenate([h, s_l], axis=1)
    feats = _matmul_bias(hs, W_att, b_att, bm=1000)
    G = _matmul_bias(feats, W1, jnp.zeros_like(b1), bm=1000)

    m = jax.nn.relu(G[i] - G[j] + b1)
    agg_up = jax.ops.segment_sum(m, jpos, num_segments=number_up)
    cnt = jax.ops.segment_sum(
        jnp.ones(jpos.shape, dtype=jnp.float32), jpos, num_segments=number_up
    )
    agg_up = agg_up / jnp.maximum(cnt, 1.0)[:, None]
    upd = _matmul_bias(agg_up, W2, jnp.zeros_like(b2), bm=640)
    out = (feats + b2).at[nodes_up].add(upd)
    return out, up_points, i, j


# speedup vs baseline: 1.6580x; 1.3246x over previous
import functools

import jax
import jax.numpy as jnp
from jax import lax
from jax.experimental import pallas as pl
from jax.experimental.pallas import tpu as pltpu
from jax.experimental.pallas import tpu_sc as plsc

_NC = 2
_NS = 16
_D = 512
_DH = _D // _NC
_EC = 64
_NCH = 37
_QC = 64
_EPT = _EC * _NCH
_NE = _NS * _EPT
_ACC_ROWS = 2560
_OWN = _ACC_ROWS // _NS
_W = 2048
_NW2 = (_NE + _W - 1) // _W
_NEW = _NW2 * _W


def _matmul_bias_kernel(x_ref, w_ref, b_ref, o_ref, *, relu):
    acc = jnp.dot(x_ref[...], w_ref[...], preferred_element_type=jnp.float32)
    acc = acc + b_ref[...]
    if relu:
        acc = jnp.maximum(acc, 0.0)
    o_ref[...] = acc


def _feats_kernel(h_ref, p_ref, w_ref, b_ref, o_ref):
    acc = jnp.dot(h_ref[...], w_ref[0:512, :], preferred_element_type=jnp.float32)
    acc = acc + jnp.dot(p_ref[...], w_ref[512:515, :],
                        preferred_element_type=jnp.float32)
    o_ref[...] = acc + b_ref[...]


def _feats_mm(h, p2, w, b, bm=1000):
    m = h.shape[0]
    n = w.shape[1]
    return pl.pallas_call(
        _feats_kernel,
        grid=(pl.cdiv(m, bm),),
        in_specs=[
            pl.BlockSpec((bm, h.shape[1]), lambda i: (i, 0)),
            pl.BlockSpec((bm, p2.shape[1]), lambda i: (i, 0)),
            pl.BlockSpec((w.shape[0], n), lambda i: (0, 0)),
            pl.BlockSpec((1, n), lambda i: (0, 0)),
        ],
        out_specs=pl.BlockSpec((bm, n), lambda i: (i, 0)),
        out_shape=jax.ShapeDtypeStruct((m, n), jnp.float32),
    )(h, p2, w, b.reshape(1, n))


def _matmul_bias(x, w, b, relu=False, bm=1024):
    m, k = x.shape
    n = w.shape[1]
    grid = (pl.cdiv(m, bm),)
    return pl.pallas_call(
        functools.partial(_matmul_bias_kernel, relu=relu),
        grid=grid,
        in_specs=[
            pl.BlockSpec((bm, k), lambda i: (i, 0)),
            pl.BlockSpec((k, n), lambda i: (0, 0)),
            pl.BlockSpec((1, n), lambda i: (0, 0)),
        ],
        out_specs=pl.BlockSpec((bm, n), lambda i: (i, 0)),
        out_shape=jax.ShapeDtypeStruct((m, n), jnp.float32),
    )(x, w, b.reshape(1, n))


def _matmul_two_kernel(x_ref, w_ref, b_ref, g_ref, a_ref):
    acc = jnp.dot(x_ref[...], w_ref[...], preferred_element_type=jnp.float32)
    g_ref[0] = acc
    a_ref[0] = acc + b_ref[...]


def _matmul_two(x, w, b, bm=1000):
    m, k = x.shape
    n = w.shape[1]
    shp = jax.ShapeDtypeStruct((_NC, m, n // _NC), jnp.float32)
    return pl.pallas_call(
        _matmul_two_kernel,
        grid=(_NC, pl.cdiv(m, bm)),
        in_specs=[
            pl.BlockSpec((bm, k), lambda hf, i: (i, 0)),
            pl.BlockSpec((k, n // _NC), lambda hf, i: (0, hf)),
            pl.BlockSpec((1, n // _NC), lambda hf, i: (0, hf)),
        ],
        out_specs=[
            pl.BlockSpec((1, bm, n // _NC), lambda hf, i: (hf, i, 0)),
            pl.BlockSpec((1, bm, n // _NC), lambda hf, i: (hf, i, 0)),
        ],
        out_shape=[shp, shp],
    )(x, w, b.reshape(1, n))


def _edge_msg_kernel(a_hbm, g_hbm, ei_hbm, ejg_hbm, m_hbm,
                     idi, idjg, buf_a0, buf_g0, buf_a1, buf_g1, sem0, sem1):
    c = lax.axis_index("c")
    s = lax.axis_index("s")

    pltpu.sync_copy(ei_hbm.at[s], idi)
    pltpu.sync_copy(ejg_hbm.at[s], idjg)

    off = (c * 10000).astype(jnp.int32) + jnp.zeros((16,), jnp.int32)

    def _reb_row(r, carry):
        for t in range(_EC // 16):
            idi[r, pl.ds(t * 16, 16)] = idi[r, pl.ds(t * 16, 16)] + off
            idjg[r, pl.ds(t * 16, 16)] = idjg[r, pl.ds(t * 16, 16)] + off
        return carry

    lax.fori_loop(0, _NCH, _reb_row, 0)

    def _issue(ci, buf_a, buf_g, sem):
        pltpu.async_copy(a_hbm.at[idi.at[ci]], buf_a, sem)
        pltpu.async_copy(g_hbm.at[idjg.at[ci]], buf_g, sem)

    def _wait(buf_a, buf_g, sem):
        pltpu.make_async_copy(a_hbm.at[idi.at[0]], buf_a, sem).wait()
        pltpu.make_async_copy(g_hbm.at[idjg.at[0]], buf_g, sem).wait()

    def _compute(ci, buf_a, buf_g):
        def _row(r, carry):
            def _vec(v, c2):
                av = buf_a[r, pl.ds(v * 16, 16)]
                gv = buf_g[r, pl.ds(v * 16, 16)]
                buf_a[r, pl.ds(v * 16, 16)] = jnp.maximum(av - gv, 0.0)
                return c2
            return lax.fori_loop(0, _DH // 16, _vec, carry)

        lax.fori_loop(0, _EC, _row, 0)
        pltpu.sync_copy(buf_a, m_hbm.at[c, pl.ds(s * _EPT + ci * _EC, _EC)])

    _issue(0, buf_a0, buf_g0, sem0)
    _issue(1, buf_a1, buf_g1, sem1)

    def _pair(k, carry):
        ci = 2 * k
        _wait(buf_a0, buf_g0, sem0)
        _compute(ci, buf_a0, buf_g0)

        @pl.when(ci + 2 < _NCH)
        def _():
            _issue(ci + 2, buf_a0, buf_g0, sem0)

        _wait(buf_a1, buf_g1, sem1)
        _compute(ci + 1, buf_a1, buf_g1)

        @pl.when(ci + 3 < _NCH)
        def _():
            _issue(ci + 3, buf_a1, buf_g1, sem1)

        return carry

    lax.fori_loop(0, _NCH // 2, _pair, 0)
    if _NCH % 2 == 1:
        _wait(buf_a0, buf_g0, sem0)
        _compute(_NCH - 1, buf_a0, buf_g0)


def _edge_msg(a2, g2, ei, ejg):
    k = pl.kernel(
        _edge_msg_kernel,
        mesh=plsc.VectorSubcoreMesh(core_axis_name="c", subcore_axis_name="s"),
        out_type=jax.ShapeDtypeStruct((_NC, _NE, _DH), jnp.float32),
        scratch_types=[
            pltpu.VMEM((_NCH, _EC), jnp.int32),
            pltpu.VMEM((_NCH, _EC), jnp.int32),
            pltpu.VMEM((_EC, _DH), jnp.float32),
            pltpu.VMEM((_EC, _DH), jnp.float32),
            pltpu.VMEM((_EC, _DH), jnp.float32),
            pltpu.VMEM((_EC, _DH), jnp.float32),
            pltpu.SemaphoreType.DMA,
            pltpu.SemaphoreType.DMA,
        ],
    )
    nflat = _NC * a2.shape[1]
    return k(a2.reshape(nflat, _DH), g2.reshape(nflat, _DH), ei, ejg)


def _seg_sum_kernel(m_hbm, es_hbm, jr_hbm, bnd_hbm, out_hbm,
                    eids, bvm, idxc0, idxc1, jrc0, jrc1, gbuf0, gbuf1,
                    acc, sem0, sem1):
    c = lax.axis_index("c")
    s = lax.axis_index("s")
    row0 = s * _OWN

    pltpu.sync_copy(bnd_hbm, bvm)
    pltpu.sync_copy(es_hbm, eids)

    start = bvm[pl.ds(s, 16)][0]
    end = bvm[pl.ds(s + 1, 16)][0]
    base = (start // _QC) * _QC
    nq = (end - base + _QC - 1) // _QC

    zf = jnp.zeros((16,), jnp.float32)

    def _zr(r, carry):
        def _zv(v, c2):
            acc[r, pl.ds(v * 16, 16)] = zf
            return c2
        return lax.fori_loop(0, _DH // 16, _zv, carry)

    lax.fori_loop(0, _OWN, _zr, 0)

    off = (c * _NE).astype(jnp.int32) + jnp.zeros((16,), jnp.int32)

    def _issue(q, idxc, jrc, gbuf, sem):
        p = base + q * _QC
        for t in range(_QC // 16):
            idxc[pl.ds(t * 16, 16)] = eids[pl.ds(p + t * 16, 16)] + off
        pltpu.async_copy(m_hbm.at[idxc], gbuf, sem)
        pltpu.async_copy(jr_hbm.at[pl.ds(p, _QC + 16)], jrc, sem)

    def _wait(jrc, gbuf, sem):
        pltpu.make_async_copy(m_hbm.at[idxc0], gbuf, sem).wait()
        pltpu.make_async_copy(jr_hbm.at[pl.ds(0, _QC + 16)], jrc, sem).wait()

    def _accum(q, jrc, gbuf, carry2):
        p = base + q * _QC
        lo = jnp.maximum(0, start - p)
        hi = jnp.minimum(_QC, end - p)

        def _acc1(k2, c3):
            rr = jrc[pl.ds(k2, 16)][0]

            def _av(v, c4):
                acc[rr, pl.ds(v * 16, 16)] = (
                    acc[rr, pl.ds(v * 16, 16)] + gbuf[k2, pl.ds(v * 16, 16)]
                )
                return c4

            return lax.fori_loop(0, _DH // 16, _av, c3)

        return lax.fori_loop(lo, hi, _acc1, carry2)

    @pl.when(nq > 0)
    def _():
        _issue(0, idxc0, jrc0, gbuf0, sem0)

    def _pair(k, carry):
        q0 = 2 * k
        q1 = q0 + 1
        _wait(jrc0, gbuf0, sem0)

        @pl.when(q1 < nq)
        def _():
            _issue(q1, idxc1, jrc1, gbuf1, sem1)

        _accum(q0, jrc0, gbuf0, 0)

        @pl.when(q0 + 2 < nq)
        def _():
            _issue(q0 + 2, idxc0, jrc0, gbuf0, sem0)

        @pl.when(q1 < nq)
        def _():
            _wait(jrc1, gbuf1, sem1)
            _accum(q1, jrc1, gbuf1, 0)

        return carry

    lax.fori_loop(0, (nq + 1) // 2, _pair, 0)
    pltpu.sync_copy(acc, out_hbm.at[c, pl.ds(row0, _OWN)])


def _seg_sum(m2, es, jr, bnd):
    k = pl.kernel(
        _seg_sum_kernel,
        mesh=plsc.VectorSubcoreMesh(core_axis_name="c", subcore_axis_name="s"),
        out_type=jax.ShapeDtypeStruct((_NC, _ACC_ROWS, _DH), jnp.float32),
        scratch_types=[
            pltpu.VMEM((_NE + 64,), jnp.int32),
            pltpu.VMEM((32,), jnp.int32),
            pltpu.VMEM((_QC,), jnp.int32),
            pltpu.VMEM((_QC,), jnp.int32),
            pltpu.VMEM((_QC + 16,), jnp.int32),
            pltpu.VMEM((_QC + 16,), jnp.int32),
            pltpu.VMEM((_QC, _DH), jnp.float32),
            pltpu.VMEM((_QC, _DH), jnp.float32),
            pltpu.VMEM((_OWN, _DH), jnp.float32),
            pltpu.SemaphoreType.DMA,
            pltpu.SemaphoreType.DMA,
        ],
    )
    return k(m2.reshape(_NC * _NE, _DH), es, jr, bnd)


def _knn_kernel(dp_ref, upt_ref, o_ref, *, nu, m_i):
    dx = dp_ref[:, 0:1]
    dy = dp_ref[:, 1:2]
    dz = dp_ref[:, 2:3]
    ux = upt_ref[0:1, :]
    uy = upt_ref[1:2, :]
    uz = upt_ref[2:3, :]
    d2 = (dx - ux) ** 2 + (dy - uy) ** 2
    d2 = d2 + (dz - uz) ** 2
    bm = d2.shape[0]
    iota = jax.lax.broadcasted_iota(jnp.int32, (bm, nu), 1)
    for k in range(m_i):
        mv = jnp.min(d2, axis=1, keepdims=True)
        sel = jnp.where(d2 == mv, iota, nu)
        mi = jnp.min(sel, axis=1, keepdims=True)
        o_ref[:, k : k + 1] = mi
        d2 = jnp.where(iota == mi, jnp.inf, d2)


def _knn(down_pos, up_pos_t, m_i, bm=768):
    nd = down_pos.shape[0]
    nu = up_pos_t.shape[1]
    return pl.pallas_call(
        functools.partial(_knn_kernel, nu=nu, m_i=m_i),
        grid=(pl.cdiv(nd, bm),),
        in_specs=[
            pl.BlockSpec((bm, 3), lambda i: (i, 0)),
            pl.BlockSpec((3, nu), lambda i: (0, 0)),
        ],
        out_specs=pl.BlockSpec((bm, m_i), lambda i: (i, 0)),
        out_shape=jax.ShapeDtypeStruct((nd, m_i), jnp.int32),
    )(down_pos, up_pos_t)


def kernel(h, s_l, scores, W_att, b_att, W1, b1, W2, b2):
    n = h.shape[0]
    number_up = n // 4
    number_down = n - number_up
    M_i = 5

    order = jnp.flip(jnp.argsort(scores))
    up_idx = order[:number_up]
    up_points = jnp.zeros((n,), dtype=bool).at[up_idx].set(True)
    cs_up = jnp.cumsum(up_points.astype(jnp.int32))
    arange_n = jnp.arange(n, dtype=jnp.int32)
    positions = jnp.where(
        up_points, number_down + cs_up - 1, arange_n - cs_up
    )
    group_order = jnp.zeros((n,), jnp.int32).at[positions].set(arange_n)
    nodes_down = group_order[:number_down]
    nodes_up = group_order[number_down:]

    down_pos = s_l[nodes_down]
    up_pos_t = s_l[nodes_up].T
    neigh = _knn(down_pos, up_pos_t, M_i)
    jpos = neigh.reshape(-1)
    j = nodes_up[jpos]
    i = jnp.repeat(nodes_down, M_i)

    feats = _feats_mm(h, s_l, W_att, b_att, bm=1000)
    G, A = _matmul_two(feats, W1, b1, bm=1000)

    n_edges = number_down * M_i
    pad = _NE - n_edges
    ei = jnp.concatenate([i, jnp.zeros((pad,), jnp.int32)]).reshape(_NS, _NCH, _EC)
    ejg = jnp.concatenate([j, jnp.zeros((pad,), jnp.int32)]).reshape(_NS, _NCH, _EC)
    jpos_p = jnp.concatenate([jpos, jnp.full((pad,), _ACC_ROWS - 1, jnp.int32)])
    bkt = jpos_p // _OWN
    oh = (bkt[:, None] == jnp.arange(_NS)[None, :]).astype(jnp.int32)
    pos_in = jnp.sum(jnp.cumsum(oh, axis=0) * oh, axis=1)
    offs = jnp.concatenate(
        [jnp.zeros((1,), jnp.int32), jnp.cumsum(jnp.sum(oh, axis=0))]
    ).astype(jnp.int32)
    position = offs[bkt] + pos_in - 1
    es = jnp.zeros((_NE + 64,), jnp.int32).at[position].set(
        jnp.arange(_NE, dtype=jnp.int32)
    )
    jr = jnp.zeros((_NE + 128,), jnp.int32).at[position].set(jpos_p - bkt * _OWN)
    bnd = jnp.concatenate([offs, jnp.full((15,), offs[_NS], jnp.int32)])
    m2 = _edge_msg(A, G, ei, ejg)
    partial = _seg_sum(m2, es, jr, bnd)

    cnt = jax.ops.segment_sum(
        jnp.ones(jpos.shape, dtype=jnp.float32), jpos, num_segments=number_up
    )
    agg_up = jnp.concatenate(
        [partial[0, :number_up], partial[1, :number_up]], axis=1
    )
    agg_up = agg_up / jnp.maximum(cnt, 1.0)[:, None]
    upd = _matmul_bias(agg_up, W2, jnp.zeros_like(b2), bm=640)
    upd_p = jnp.concatenate([upd, jnp.zeros((1, _D), jnp.float32)])
    idxg = jnp.where(up_points, cs_up - 1, number_up)
    out = feats + b2 + upd_p[idxg]
    return out, up_points, i, j


# speedup vs baseline: 1.6580x; 1.0000x over previous
import functools

import jax
import jax.numpy as jnp
from jax import lax
from jax.experimental import pallas as pl
from jax.experimental.pallas import tpu as pltpu
from jax.experimental.pallas import tpu_sc as plsc

_NC = 2
_NS = 16
_D = 512
_DH = _D // _NC
_EC = 64
_NCH = 37
_QC = 64
_EPT = _EC * _NCH
_NE = _NS * _EPT
_ACC_ROWS = 2560
_OWN = _ACC_ROWS // _NS


def _matmul_bias_kernel(x_ref, w_ref, b_ref, o_ref, *, relu):
    acc = jnp.dot(x_ref[...], w_ref[...], preferred_element_type=jnp.float32)
    acc = acc + b_ref[...]
    if relu:
        acc = jnp.maximum(acc, 0.0)
    o_ref[...] = acc


def _feats_kernel(h_ref, p_ref, w_ref, b_ref, o_ref):
    acc = jnp.dot(h_ref[...], w_ref[0:512, :], preferred_element_type=jnp.float32)
    acc = acc + jnp.dot(p_ref[...], w_ref[512:515, :],
                        preferred_element_type=jnp.float32)
    o_ref[...] = acc + b_ref[...]


def _feats_mm(h, p2, w, b, bm=1000):
    m = h.shape[0]
    n = w.shape[1]
    return pl.pallas_call(
        _feats_kernel,
        grid=(pl.cdiv(m, bm),),
        in_specs=[
            pl.BlockSpec((bm, h.shape[1]), lambda i: (i, 0)),
            pl.BlockSpec((bm, p2.shape[1]), lambda i: (i, 0)),
            pl.BlockSpec((w.shape[0], n), lambda i: (0, 0)),
            pl.BlockSpec((1, n), lambda i: (0, 0)),
        ],
        out_specs=pl.BlockSpec((bm, n), lambda i: (i, 0)),
        out_shape=jax.ShapeDtypeStruct((m, n), jnp.float32),
    )(h, p2, w, b.reshape(1, n))


def _matmul_bias(x, w, b, relu=False, bm=1024):
    m, k = x.shape
    n = w.shape[1]
    grid = (pl.cdiv(m, bm),)
    return pl.pallas_call(
        functools.partial(_matmul_bias_kernel, relu=relu),
        grid=grid,
        in_specs=[
            pl.BlockSpec((bm, k), lambda i: (i, 0)),
            pl.BlockSpec((k, n), lambda i: (0, 0)),
            pl.BlockSpec((1, n), lambda i: (0, 0)),
        ],
        out_specs=pl.BlockSpec((bm, n), lambda i: (i, 0)),
        out_shape=jax.ShapeDtypeStruct((m, n), jnp.float32),
    )(x, w, b.reshape(1, n))


def _matmul_two_kernel(x_ref, w_ref, b_ref, g_ref, a_ref):
    acc = jnp.dot(x_ref[...], w_ref[...], preferred_element_type=jnp.float32)
    g_ref[0] = acc
    a_ref[0] = acc + b_ref[...]


def _matmul_two(x, w, b, bm=1000):
    m, k = x.shape
    n = w.shape[1]
    shp = jax.ShapeDtypeStruct((_NC, m, n // _NC), jnp.float32)
    return pl.pallas_call(
        _matmul_two_kernel,
        grid=(_NC, pl.cdiv(m, bm)),
        in_specs=[
            pl.BlockSpec((bm, k), lambda hf, i: (i, 0)),
            pl.BlockSpec((k, n // _NC), lambda hf, i: (0, hf)),
            pl.BlockSpec((1, n // _NC), lambda hf, i: (0, hf)),
        ],
        out_specs=[
            pl.BlockSpec((1, bm, n // _NC), lambda hf, i: (hf, i, 0)),
            pl.BlockSpec((1, bm, n // _NC), lambda hf, i: (hf, i, 0)),
        ],
        out_shape=[shp, shp],
    )(x, w, b.reshape(1, n))


def _edge_msg_kernel(a_hbm, g_hbm, ei_hbm, ejg_hbm, m_hbm,
                     idi, idjg, buf_a0, buf_g0, buf_a1, buf_g1, sem0, sem1):
    c = lax.axis_index("c")
    s = lax.axis_index("s")

    pltpu.sync_copy(ei_hbm.at[s], idi)
    pltpu.sync_copy(ejg_hbm.at[s], idjg)

    off = (c * 10000).astype(jnp.int32) + jnp.zeros((16,), jnp.int32)

    def _reb_row(r, carry):
        for t in range(_EC // 16):
            idi[r, pl.ds(t * 16, 16)] = idi[r, pl.ds(t * 16, 16)] + off
            idjg[r, pl.ds(t * 16, 16)] = idjg[r, pl.ds(t * 16, 16)] + off
        return carry

    lax.fori_loop(0, _NCH, _reb_row, 0)

    def _issue(ci, buf_a, buf_g, sem):
        pltpu.async_copy(a_hbm.at[idi.at[ci]], buf_a, sem)
        pltpu.async_copy(g_hbm.at[idjg.at[ci]], buf_g, sem)

    def _wait(buf_a, buf_g, sem):
        pltpu.make_async_copy(a_hbm.at[idi.at[0]], buf_a, sem).wait()
        pltpu.make_async_copy(g_hbm.at[idjg.at[0]], buf_g, sem).wait()

    def _compute(ci, buf_a, buf_g):
        def _row(r, carry):
            def _vec(v, c2):
                av = buf_a[r, pl.ds(v * 16, 16)]
                gv = buf_g[r, pl.ds(v * 16, 16)]
                buf_a[r, pl.ds(v * 16, 16)] = jnp.maximum(av - gv, 0.0)
                return c2
            return lax.fori_loop(0, _DH // 16, _vec, carry)

        lax.fori_loop(0, _EC, _row, 0)
        pltpu.sync_copy(buf_a, m_hbm.at[c, pl.ds(s * _EPT + ci * _EC, _EC)])

    _issue(0, buf_a0, buf_g0, sem0)
    _issue(1, buf_a1, buf_g1, sem1)

    def _pair(k, carry):
        ci = 2 * k
        _wait(buf_a0, buf_g0, sem0)
        _compute(ci, buf_a0, buf_g0)

        @pl.when(ci + 2 < _NCH)
        def _():
            _issue(ci + 2, buf_a0, buf_g0, sem0)

        _wait(buf_a1, buf_g1, sem1)
        _compute(ci + 1, buf_a1, buf_g1)

        @pl.when(ci + 3 < _NCH)
        def _():
            _issue(ci + 3, buf_a1, buf_g1, sem1)

        return carry

    lax.fori_loop(0, _NCH // 2, _pair, 0)
    if _NCH % 2 == 1:
        _wait(buf_a0, buf_g0, sem0)
        _compute(_NCH - 1, buf_a0, buf_g0)


def _edge_msg(a2, g2, ei, ejg):
    k = pl.kernel(
        _edge_msg_kernel,
        mesh=plsc.VectorSubcoreMesh(core_axis_name="c", subcore_axis_name="s"),
        out_type=jax.ShapeDtypeStruct((_NC, _NE, _DH), jnp.float32),
        scratch_types=[
            pltpu.VMEM((_NCH, _EC), jnp.int32),
            pltpu.VMEM((_NCH, _EC), jnp.int32),
            pltpu.VMEM((_EC, _DH), jnp.float32),
            pltpu.VMEM((_EC, _DH), jnp.float32),
            pltpu.VMEM((_EC, _DH), jnp.float32),
            pltpu.VMEM((_EC, _DH), jnp.float32),
            pltpu.SemaphoreType.DMA,
            pltpu.SemaphoreType.DMA,
        ],
    )
    nflat = _NC * a2.shape[1]
    return k(a2.reshape(nflat, _DH), g2.reshape(nflat, _DH), ei, ejg)


def _seg_sum_kernel(m_hbm, es_hbm, jr_hbm, bnd_hbm, out_hbm,
                    eids, bvm, idxc0, idxc1, jrc0, jrc1, gbuf0, gbuf1,
                    acc, sem0, sem1):
    c = lax.axis_index("c")
    s = lax.axis_index("s")
    row0 = s * _OWN

    pltpu.sync_copy(bnd_hbm, bvm)
    pltpu.sync_copy(es_hbm, eids)

    start = bvm[pl.ds(s, 16)][0]
    end = bvm[pl.ds(s + 1, 16)][0]
    base = (start // _QC) * _QC
    nq = (end - base + _QC - 1) // _QC

    zf = jnp.zeros((16,), jnp.float32)

    def _zr(r, carry):
        def _zv(v, c2):
            acc[r, pl.ds(v * 16, 16)] = zf
            return c2
        return lax.fori_loop(0, _DH // 16, _zv, carry)

    lax.fori_loop(0, _OWN, _zr, 0)

    off = (c * _NE).astype(jnp.int32) + jnp.zeros((16,), jnp.int32)

    def _issue(q, idxc, jrc, gbuf, sem):
        p = base + q * _QC
        for t in range(_QC // 16):
            idxc[pl.ds(t * 16, 16)] = eids[pl.ds(p + t * 16, 16)] + off
        pltpu.async_copy(m_hbm.at[idxc], gbuf, sem)
        pltpu.async_copy(jr_hbm.at[pl.ds(p, _QC + 16)], jrc, sem)

    def _wait(jrc, gbuf, sem):
        pltpu.make_async_copy(m_hbm.at[idxc0], gbuf, sem).wait()
        pltpu.make_async_copy(jr_hbm.at[pl.ds(0, _QC + 16)], jrc, sem).wait()

    def _accum(q, jrc, gbuf, carry2):
        p = base + q * _QC
        lo = jnp.maximum(0, start - p)
        hi = jnp.minimum(_QC, end - p)

        def _acc1(k2, c3):
            rr = jrc[pl.ds(k2, 16)][0]

            def _av(v, c4):
                acc[rr, pl.ds(v * 16, 16)] = (
                    acc[rr, pl.ds(v * 16, 16)] + gbuf[k2, pl.ds(v * 16, 16)]
                )
                return c4

            return lax.fori_loop(0, _DH // 16, _av, c3)

        return lax.fori_loop(lo, hi, _acc1, carry2)

    @pl.when(nq > 0)
    def _():
        _issue(0, idxc0, jrc0, gbuf0, sem0)

    def _pair(k, carry):
        q0 = 2 * k
        q1 = q0 + 1
        _wait(jrc0, gbuf0, sem0)

        @pl.when(q1 < nq)
        def _():
            _issue(q1, idxc1, jrc1, gbuf1, sem1)

        _accum(q0, jrc0, gbuf0, 0)

        @pl.when(q0 + 2 < nq)
        def _():
            _issue(q0 + 2, idxc0, jrc0, gbuf0, sem0)

        @pl.when(q1 < nq)
        def _():
            _wait(jrc1, gbuf1, sem1)
            _accum(q1, jrc1, gbuf1, 0)

        return carry

    lax.fori_loop(0, (nq + 1) // 2, _pair, 0)
    pltpu.sync_copy(acc, out_hbm.at[c, pl.ds(row0, _OWN)])


def _seg_sum(m2, es, jr, bnd):
    k = pl.kernel(
        _seg_sum_kernel,
        mesh=plsc.VectorSubcoreMesh(core_axis_name="c", subcore_axis_name="s"),
        out_type=jax.ShapeDtypeStruct((_NC, _ACC_ROWS, _DH), jnp.float32),
        scratch_types=[
            pltpu.VMEM((_NE + 64,), jnp.int32),
            pltpu.VMEM((32,), jnp.int32),
            pltpu.VMEM((_QC,), jnp.int32),
            pltpu.VMEM((_QC,), jnp.int32),
            pltpu.VMEM((_QC + 16,), jnp.int32),
            pltpu.VMEM((_QC + 16,), jnp.int32),
            pltpu.VMEM((_QC, _DH), jnp.float32),
            pltpu.VMEM((_QC, _DH), jnp.float32),
            pltpu.VMEM((_OWN, _DH), jnp.float32),
            pltpu.SemaphoreType.DMA,
            pltpu.SemaphoreType.DMA,
        ],
    )
    return k(m2.reshape(_NC * _NE, _DH), es, jr, bnd)


def _knn_kernel(dp_ref, upt_ref, o_ref, *, nu, m_i):
    dx = dp_ref[:, 0:1]
    dy = dp_ref[:, 1:2]
    dz = dp_ref[:, 2:3]
    ux = upt_ref[0:1, :]
    uy = upt_ref[1:2, :]
    uz = upt_ref[2:3, :]
    d2 = (dx - ux) ** 2 + (dy - uy) ** 2
    d2 = d2 + (dz - uz) ** 2
    bm = d2.shape[0]
    iota = jax.lax.broadcasted_iota(jnp.int32, (bm, nu), 1)
    for k in range(m_i):
        mv = jnp.min(d2, axis=1, keepdims=True)
        sel = jnp.where(d2 == mv, iota, nu)
        mi = jnp.min(sel, axis=1, keepdims=True)
        o_ref[:, k : k + 1] = mi
        d2 = jnp.where(iota == mi, jnp.inf, d2)


def _knn(down_pos, up_pos_t, m_i, bm=768):
    nd = down_pos.shape[0]
    nu = up_pos_t.shape[1]
    return pl.pallas_call(
        functools.partial(_knn_kernel, nu=nu, m_i=m_i),
        grid=(pl.cdiv(nd, bm),),
        in_specs=[
            pl.BlockSpec((bm, 3), lambda i: (i, 0)),
            pl.BlockSpec((3, nu), lambda i: (0, 0)),
        ],
        out_specs=pl.BlockSpec((bm, m_i), lambda i: (i, 0)),
        out_shape=jax.ShapeDtypeStruct((nd, m_i), jnp.int32),
    )(down_pos, up_pos_t)


def kernel(h, s_l, scores, W_att, b_att, W1, b1, W2, b2):
    n = h.shape[0]
    number_up = n // 4
    number_down = n - number_up
    M_i = 5

    order = jnp.flip(jnp.argsort(scores))
    up_idx = order[:number_up]
    up_points = jnp.zeros((n,), dtype=bool).at[up_idx].set(True)
    cs_up = jnp.cumsum(up_points.astype(jnp.int32))
    arange_n = jnp.arange(n, dtype=jnp.int32)
    positions = jnp.where(
        up_points, number_down + cs_up - 1, arange_n - cs_up
    )
    group_order = jnp.zeros((n,), jnp.int32).at[positions].set(arange_n)
    nodes_down = group_order[:number_down]
    nodes_up = group_order[number_down:]

    down_pos = s_l[nodes_down]
    up_pos_t = s_l[nodes_up].T
    neigh = _knn(down_pos, up_pos_t, M_i)
    jpos = neigh.reshape(-1)
    j = nodes_up[jpos]
    i = jnp.repeat(nodes_down, M_i)

    feats = _feats_mm(h, s_l, W_att, b_att, bm=1000)
    G, A = _matmul_two(feats, W1, b1, bm=1000)

    n_edges = number_down * M_i
    pad = _NE - n_edges
    ei = jnp.concatenate([i, jnp.zeros((pad,), jnp.int32)]).reshape(_NS, _NCH, _EC)
    ejg = jnp.concatenate([j, jnp.zeros((pad,), jnp.int32)]).reshape(_NS, _NCH, _EC)
    jpos_p = jnp.concatenate([jpos, jnp.full((pad,), _ACC_ROWS - 1, jnp.int32)])
    bkt = jpos_p // _OWN
    oh = (bkt[:, None] == jnp.arange(_NS)[None, :]).astype(jnp.int32)
    pos_in = jnp.sum(jnp.cumsum(oh, axis=0) * oh, axis=1)
    offs = jnp.concatenate(
        [jnp.zeros((1,), jnp.int32), jnp.cumsum(jnp.sum(oh, axis=0))]
    ).astype(jnp.int32)
    position = offs[bkt] + pos_in - 1
    es = jnp.zeros((_NE + 64,), jnp.int32).at[position].set(
        jnp.arange(_NE, dtype=jnp.int32)
    )
    jr = jnp.zeros((_NE + 128,), jnp.int32).at[position].set(jpos_p - bkt * _OWN)
    bnd = jnp.concatenate([offs, jnp.full((15,), offs[_NS], jnp.int32)])
    m2 = _edge_msg(A, G, ei, ejg)
    partial = _seg_sum(m2, es, jr, bnd)

    cnt = jax.ops.segment_sum(
        jnp.ones(jpos.shape, dtype=jnp.float32), jpos, num_segments=number_up
    )
    agg_up = jnp.concatenate(
        [partial[0, :number_up], partial[1, :number_up]], axis=1
    )
    agg_up = agg_up / jnp.maximum(cnt, 1.0)[:, None]
    upd = _matmul_bias(agg_up, W2, jnp.zeros_like(b2), bm=640)
    upd_p = jnp.concatenate([upd, jnp.zeros((1, _D), jnp.float32)])
    idxg = jnp.where(up_points, cs_up - 1, number_up)
    out = feats + b2 + upd_p[idxg]
    return out, up_points, i, j
